# Initial kernel scaffold; baseline (speedup 1.0000x reference)
import functools

import jax
import jax.numpy as jnp
from jax import lax
from jax.experimental import pallas as pl
from jax.experimental.pallas import tpu as pltpu

P = 32768
B = 16
T = 4096
F_IN = 16
D = 128
H = 256
N_MORPH = 6
LOG_THRESH = 4.0

C = 4096
G = P // C

_F32 = jnp.float32
_HIGH = lax.Precision.HIGHEST


def _dotT(x, y):
    return lax.dot_general(x, y, (((0,), (0,)), ((), ())),
                           precision=_HIGH, preferred_element_type=_F32)


def _dot(x, y):
    return lax.dot_general(x, y, (((1,), (0,)), ((), ())),
                           precision=_HIGH, preferred_element_type=_F32)


def _body(feats_ref, coords_ref, ids_ref,
          wb1, bb1, wb2, bb2,
          wm1d, wm1c, bm1, wm2, bm2,
          wec1d, wec1c, bec1, wec2, bec2,
          weu1d, weu1c, beu1, weu2, beu2,
          wdc1d, wdc1c, bdc1, wdc2, bdc2,
          wdl1d, wdl1c, bdl1, wdl2, bdl2,
          wdh1d, wdh1c, bdh1, wdh2, bdh2,
          out_ref, acc_a, acc_cn, cnt):
    g = pl.program_id(0)

    @pl.when(g == 0)
    def _init():
        acc_a[...] = jnp.zeros_like(acc_a)
        acc_cn[...] = jnp.zeros_like(acc_cn)
        cnt[...] = jnp.zeros_like(cnt)

    ids = ids_ref[...]
    iota_b = lax.broadcasted_iota(jnp.int32, (C, B), 1)
    onehot = (ids == iota_b).astype(_F32)
    ltmat = (ids < iota_b).astype(_F32)
    ones_c = jnp.ones((C, 1), _F32)

    cnt_lt = _dotT(ltmat, ones_c)
    pos = _dot(onehot, cnt[...] - cnt_lt) \
        + lax.broadcasted_iota(_F32, (C, 1), 0)
    keep = (pos < T).astype(_F32)
    om = onehot * keep

    a = jnp.maximum(_dot(feats_ref[...], wb1[...]) + bb1[...], 0.0)
    acc_a[...] += _dotT(om, a)
    xe = jnp.concatenate([coords_ref[...], ones_c], axis=1)
    acc_cn[...] += _dotT(om, xe)
    cnt[...] += _dotT(onehot, ones_c)

    @pl.when(g == G - 1)
    def _final():
        n = acc_cn[:, 3:4]
        inv = 1.0 / jnp.maximum(n, 1.0)
        pt = (_dot(acc_a[...], wb2[...]) + n * bb2[...]) * inv
        pc = acc_cn[:, 0:3] * inv

        def head(w1d, w1c, b1, w2, b2):
            h = jnp.maximum(_dot(pt, w1d[...]) + _dot(pc, w1c[...]) + b1[...], 0.0)
            return _dot(h, w2[...]) + b2[...]

        ml = head(wm1d, wm1c, bm1, wm2, bm2)
        mx = jnp.max(ml, axis=-1, keepdims=True)
        ex = jnp.exp(ml - mx)
        probs = jnp.maximum(ex / jnp.sum(ex, axis=-1, keepdims=True), 1e-6)
        p_cont = probs[:, 0:2].sum(-1, keepdims=True)
        p_uncont = probs[:, 2:4].sum(-1, keepdims=True) + probs[:, 5:6]
        e_cont = head(wec1d, wec1c, bec1, wec2, bec2)
        e_uncont = head(weu1d, weu1c, beu1, weu2, beu2)
        energy = p_cont * e_cont + p_uncont * e_uncont
        p_cas = probs[:, 0:1]
        p_track = probs[:, 1:4].sum(-1, keepdims=True) + probs[:, 5:6]
        gate = 1.0 / (1.0 + jnp.exp(LOG_THRESH - energy[:, 0:1]))
        d_cas = head(wdc1d, wdc1c, bdc1, wdc2, bdc2)
        d_low = head(wdl1d, wdl1c, bdl1, wdl2, bdl2)
        d_high = head(wdh1d, wdh1c, bdh1, wdh2, bdh2)
        dirp = (p_cas * d_cas + p_track * (1.0 - gate) * d_low
                + p_track * gate * d_high)
        out_ref[...] = jnp.concatenate([ml, energy, dirp], axis=-1)


def _chunk_spec(width):
    return pl.BlockSpec((C, width), lambda g: (g, 0))


def _const_spec(shape):
    return pl.BlockSpec(shape, lambda g: tuple(0 for _ in shape))


@functools.partial(jax.jit, static_argnames=("interpret",))
def _run(coords, features, ids2, *weights, interpret=False):
    wspecs = [_const_spec(w.shape) for w in weights]
    return pl.pallas_call(
        _body,
        grid=(G,),
        in_specs=[_chunk_spec(F_IN), _chunk_spec(3), _chunk_spec(1)] + wspecs,
        out_specs=_const_spec((B, 11)),
        out_shape=jax.ShapeDtypeStruct((B, 11), _F32),
        scratch_shapes=[pltpu.VMEM((B, D), _F32),
                        pltpu.VMEM((B, 4), _F32),
                        pltpu.VMEM((B, 1), _F32)],
        interpret=interpret,
    )(features, coords, ids2, *weights)


def kernel(coords, features, batch_ids, Wb1, bb1, Wb2, bb2, Wm1, bm1, Wm2, bm2,
           Wec1, bec1, Wec2, bec2, Weu1, beu1, Weu2, beu2,
           Wdc1, bdc1, Wdc2, bdc2, Wdl1, bdl1, Wdl2, bdl2,
           Wdh1, bdh1, Wdh2, bdh2, interpret=False):
    ids2 = batch_ids.astype(jnp.int32).reshape(P, 1)
    row = lambda b: b.reshape(1, -1)
    weights = (Wb1, row(bb1), Wb2, row(bb2),
               Wm1[:D], Wm1[D:], row(bm1), Wm2, row(bm2),
               Wec1[:D], Wec1[D:], row(bec1), Wec2, row(bec2),
               Weu1[:D], Weu1[D:], row(beu1), Weu2, row(beu2),
               Wdc1[:D], Wdc1[D:], row(bdc1), Wdc2, row(bdc2),
               Wdl1[:D], Wdl1[D:], row(bdl1), Wdl2, row(bdl2),
               Wdh1[:D], Wdh1[D:], row(bdh1), Wdh2, row(bdh2))
    return _run(coords, features, ids2, *weights, interpret=interpret)


# speedup vs baseline: 3.5397x; 3.5397x over previous
import functools

import jax
import jax.numpy as jnp
from jax import lax
from jax.experimental import pallas as pl
from jax.experimental.pallas import tpu as pltpu

P = 32768
B = 16
T = 4096
F_IN = 16
D = 128
H = 256
N_MORPH = 6
LOG_THRESH = 4.0

C = 4096
G = P // C

_F32 = jnp.float32
_HIGH = lax.Precision.HIGHEST


def _dotT(x, y):
    return lax.dot_general(x, y, (((0,), (0,)), ((), ())),
                           precision=_HIGH, preferred_element_type=_F32)


def _dot(x, y):
    return lax.dot_general(x, y, (((1,), (0,)), ((), ())),
                           precision=_HIGH, preferred_element_type=_F32)


def _body(feats_ref, coords_ref, ids_ref,
          wb1, bb1, wb2, bb2,
          wm1d, wm1c, bm1, wm2, bm2,
          wec1d, wec1c, bec1, wec2, bec2,
          weu1d, weu1c, beu1, weu2, beu2,
          wdc1d, wdc1c, bdc1, wdc2, bdc2,
          wdl1d, wdl1c, bdl1, wdl2, bdl2,
          wdh1d, wdh1c, bdh1, wdh2, bdh2,
          out_ref, acc_a, acc_cn, cnt):
    g = pl.program_id(0)

    @pl.when(g == 0)
    def _init():
        acc_a[...] = jnp.zeros_like(acc_a)
        acc_cn[...] = jnp.zeros_like(acc_cn)
        cnt[...] = jnp.zeros_like(cnt)

    ids = ids_ref[...]
    iota_b = lax.broadcasted_iota(jnp.int32, (C, B), 1)
    onehot = (ids == iota_b).astype(_F32)
    ltmat = (ids < iota_b).astype(_F32)
    ones_c = jnp.ones((C, 1), _F32)

    cnt_lt = _dotT(ltmat, ones_c)
    pos = _dot(onehot, cnt[...] - cnt_lt) \
        + lax.broadcasted_iota(jnp.int32, (C, 1), 0).astype(_F32)
    keep = (pos < T).astype(_F32)
    om = onehot * keep

    a = jnp.maximum(_dot(feats_ref[...], wb1[...]) + bb1[...], 0.0)
    acc_a[...] += _dotT(om, a)
    xe = jnp.concatenate([coords_ref[...], ones_c], axis=1)
    acc_cn[...] += _dotT(om, xe)
    cnt[...] += _dotT(onehot, ones_c)

    @pl.when(g == G - 1)
    def _final():
        n = acc_cn[:, 3:4]
        inv = 1.0 / jnp.maximum(n, 1.0)
        pt = (_dot(acc_a[...], wb2[...]) + n * bb2[...]) * inv
        pc = acc_cn[:, 0:3] * inv

        def head(w1d, w1c, b1, w2, b2):
            h = jnp.maximum(_dot(pt, w1d[...]) + _dot(pc, w1c[...]) + b1[...], 0.0)
            return _dot(h, w2[...]) + b2[...]

        ml = head(wm1d, wm1c, bm1, wm2, bm2)
        mx = jnp.max(ml, axis=-1, keepdims=True)
        ex = jnp.exp(ml - mx)
        probs = jnp.maximum(ex / jnp.sum(ex, axis=-1, keepdims=True), 1e-6)
        p_cont = probs[:, 0:2].sum(-1, keepdims=True)
        p_uncont = probs[:, 2:4].sum(-1, keepdims=True) + probs[:, 5:6]
        e_cont = head(wec1d, wec1c, bec1, wec2, bec2)
        e_uncont = head(weu1d, weu1c, beu1, weu2, beu2)
        energy = p_cont * e_cont + p_uncont * e_uncont
        p_cas = probs[:, 0:1]
        p_track = probs[:, 1:4].sum(-1, keepdims=True) + probs[:, 5:6]
        gate = 1.0 / (1.0 + jnp.exp(LOG_THRESH - energy[:, 0:1]))
        d_cas = head(wdc1d, wdc1c, bdc1, wdc2, bdc2)
        d_low = head(wdl1d, wdl1c, bdl1, wdl2, bdl2)
        d_high = head(wdh1d, wdh1c, bdh1, wdh2, bdh2)
        dirp = (p_cas * d_cas + p_track * (1.0 - gate) * d_low
                + p_track * gate * d_high)
        out_ref[...] = jnp.concatenate([ml, energy, dirp], axis=-1)


def _chunk_spec(width):
    return pl.BlockSpec((C, width), lambda g: (g, 0))


def _const_spec(shape):
    return pl.BlockSpec(shape, lambda g: tuple(0 for _ in shape))


@functools.partial(jax.jit, static_argnames=("interpret",))
def _run(coords, features, ids2, *weights, interpret=False):
    wspecs = [_const_spec(w.shape) for w in weights]
    return pl.pallas_call(
        _body,
        grid=(G,),
        in_specs=[_chunk_spec(F_IN), _chunk_spec(3), _chunk_spec(1)] + wspecs,
        out_specs=_const_spec((B, 11)),
        out_shape=jax.ShapeDtypeStruct((B, 11), _F32),
        scratch_shapes=[pltpu.VMEM((B, D), _F32),
                        pltpu.VMEM((B, 4), _F32),
                        pltpu.VMEM((B, 1), _F32)],
        interpret=interpret,
    )(features, coords, ids2, *weights)


def kernel(coords, features, batch_ids, Wb1, bb1, Wb2, bb2, Wm1, bm1, Wm2, bm2,
           Wec1, bec1, Wec2, bec2, Weu1, beu1, Weu2, beu2,
           Wdc1, bdc1, Wdc2, bdc2, Wdl1, bdl1, Wdl2, bdl2,
           Wdh1, bdh1, Wdh2, bdh2, interpret=False):
    ids2 = batch_ids.astype(jnp.int32).reshape(P, 1)
    row = lambda b: b.reshape(1, -1)
    weights = (Wb1, row(bb1), Wb2, row(bb2),
               Wm1[:D], Wm1[D:], row(bm1), Wm2, row(bm2),
               Wec1[:D], Wec1[D:], row(bec1), Wec2, row(bec2),
               Weu1[:D], Weu1[D:], row(beu1), Weu2, row(beu2),
               Wdc1[:D], Wdc1[D:], row(bdc1), Wdc2, row(bdc2),
               Wdl1[:D], Wdl1[D:], row(bdl1), Wdl2, row(bdl2),
               Wdh1[:D], Wdh1[D:], row(bdh1), Wdh2, row(bdh2))
    return _run(coords, features, ids2, *weights, interpret=interpret)


# speedup vs baseline: 3.7942x; 1.0719x over previous
import functools

import jax
import jax.numpy as jnp
from jax import lax
from jax.experimental import pallas as pl
from jax.experimental.pallas import tpu as pltpu

P = 32768
B = 16
T = 4096
F_IN = 16
D = 128
H = 256
N_MORPH = 6
LOG_THRESH = 4.0

C = 4096
G = P // C

_F32 = jnp.float32
_HIGH = lax.Precision.HIGHEST


def _dotT(x, y):
    return lax.dot_general(x, y, (((0,), (0,)), ((), ())),
                           precision=_HIGH, preferred_element_type=_F32)


def _dot(x, y):
    return lax.dot_general(x, y, (((1,), (0,)), ((), ())),
                           precision=_HIGH, preferred_element_type=_F32)


def _body(feats_ref, coords_ref, ids_ref, ids_prev_ref,
          wb1, bb1, wb2, bb2,
          wm1d, wm1c, bm1, wm2, bm2,
          wec1d, wec1c, bec1, wec2, bec2,
          weu1d, weu1c, beu1, weu2, beu2,
          wdc1d, wdc1c, bdc1, wdc2, bdc2,
          wdl1d, wdl1c, bdl1, wdl2, bdl2,
          wdh1d, wdh1c, bdh1, wdh2, bdh2,
          out_ref, acc):
    g = pl.program_id(0)

    @pl.when(g == 0)
    def _init():
        acc[...] = jnp.zeros_like(acc)

    ids = ids_ref[...]
    iota_b = lax.broadcasted_iota(jnp.int32, (C, B), 1)
    onehot = (ids == iota_b).astype(_F32)

    if G > T // C:
        dropped = ids == ids_prev_ref[...]
        keep = jnp.where(g >= T // C, 1.0 - dropped.astype(_F32),
                         jnp.ones((C, 1), _F32))
    else:
        keep = jnp.ones((C, 1), _F32)

    a = jnp.maximum(_dot(feats_ref[...], wb1[...]) + bb1[...], 0.0)
    y = jnp.concatenate([a, coords_ref[...], jnp.ones((C, 1), _F32)],
                        axis=1) * keep
    acc[...] += _dotT(onehot, y)

    @pl.when(g == G - 1)
    def _final():
        n = acc[:, D + 3:D + 4]
        inv = 1.0 / jnp.maximum(n, 1.0)
        pt = (_dot(acc[:, :D], wb2[...]) + n * bb2[...]) * inv
        pc = acc[:, D:D + 3] * inv

        def head(w1d, w1c, b1, w2, b2):
            h = jnp.maximum(_dot(pt, w1d[...]) + _dot(pc, w1c[...]) + b1[...], 0.0)
            return _dot(h, w2[...]) + b2[...]

        ml = head(wm1d, wm1c, bm1, wm2, bm2)
        mx = jnp.max(ml, axis=-1, keepdims=True)
        ex = jnp.exp(ml - mx)
        probs = jnp.maximum(ex / jnp.sum(ex, axis=-1, keepdims=True), 1e-6)
        p_cont = probs[:, 0:2].sum(-1, keepdims=True)
        p_uncont = probs[:, 2:4].sum(-1, keepdims=True) + probs[:, 5:6]
        e_cont = head(wec1d, wec1c, bec1, wec2, bec2)
        e_uncont = head(weu1d, weu1c, beu1, weu2, beu2)
        energy = p_cont * e_cont + p_uncont * e_uncont
        p_cas = probs[:, 0:1]
        p_track = probs[:, 1:4].sum(-1, keepdims=True) + probs[:, 5:6]
        gate = 1.0 / (1.0 + jnp.exp(LOG_THRESH - energy[:, 0:1]))
        d_cas = head(wdc1d, wdc1c, bdc1, wdc2, bdc2)
        d_low = head(wdl1d, wdl1c, bdl1, wdl2, bdl2)
        d_high = head(wdh1d, wdh1c, bdh1, wdh2, bdh2)
        dirp = (p_cas * d_cas + p_track * (1.0 - gate) * d_low
                + p_track * gate * d_high)
        out_ref[...] = jnp.concatenate([ml, energy, dirp], axis=-1)


def _chunk_spec(width):
    return pl.BlockSpec((C, width), lambda g: (g, 0))


def _const_spec(shape):
    return pl.BlockSpec(shape, lambda g: tuple(0 for _ in shape))


@functools.partial(jax.jit, static_argnames=("interpret",))
def _run(coords, features, ids2, *weights, interpret=False):
    wspecs = [_const_spec(w.shape) for w in weights]
    prev_spec = pl.BlockSpec((C, 1), lambda g: (jnp.maximum(g - T // C, 0), 0))
    return pl.pallas_call(
        _body,
        grid=(G,),
        in_specs=[_chunk_spec(F_IN), _chunk_spec(3), _chunk_spec(1), prev_spec]
        + wspecs,
        out_specs=_const_spec((B, 11)),
        out_shape=jax.ShapeDtypeStruct((B, 11), _F32),
        scratch_shapes=[pltpu.VMEM((B, D + 4), _F32)],
        interpret=interpret,
    )(features, coords, ids2, ids2, *weights)


def kernel(coords, features, batch_ids, Wb1, bb1, Wb2, bb2, Wm1, bm1, Wm2, bm2,
           Wec1, bec1, Wec2, bec2, Weu1, beu1, Weu2, beu2,
           Wdc1, bdc1, Wdc2, bdc2, Wdl1, bdl1, Wdl2, bdl2,
           Wdh1, bdh1, Wdh2, bdh2, interpret=False):
    ids2 = batch_ids.astype(jnp.int32).reshape(P, 1)
    row = lambda b: b.reshape(1, -1)
    weights = (Wb1, row(bb1), Wb2, row(bb2),
               Wm1[:D], Wm1[D:], row(bm1), Wm2, row(bm2),
               Wec1[:D], Wec1[D:], row(bec1), Wec2, row(bec2),
               Weu1[:D], Weu1[D:], row(beu1), Weu2, row(beu2),
               Wdc1[:D], Wdc1[D:], row(bdc1), Wdc2, row(bdc2),
               Wdl1[:D], Wdl1[D:], row(bdl1), Wdl2, row(bdl2),
               Wdh1[:D], Wdh1[D:], row(bdh1), Wdh2, row(bdh2))
    return _run(coords, features, ids2, *weights, interpret=interpret)


# speedup vs baseline: 4.4781x; 1.1803x over previous
import functools

import jax
import jax.numpy as jnp
from jax import lax
from jax.experimental import pallas as pl
from jax.experimental.pallas import tpu as pltpu

P = 32768
B = 16
T = 4096
F_IN = 16
D = 128
H = 256
N_MORPH = 6
LOG_THRESH = 4.0

C = 4096
G = P // C

_F32 = jnp.float32
_HIGH = lax.Precision.HIGHEST


def _dotT(x, y):
    return lax.dot_general(x, y, (((0,), (0,)), ((), ())),
                           precision=_HIGH, preferred_element_type=_F32)


def _dot(x, y):
    return lax.dot_general(x, y, (((1,), (0,)), ((), ())),
                           precision=_HIGH, preferred_element_type=_F32)


def _body(feats_ref, coords_ref, ids_ref, ids_prev_ref,
          wb1, bb1, wb2, bb2,
          wm1, bm1, wm2, bm2,
          wec1, bec1, wec2, bec2,
          weu1, beu1, weu2, beu2,
          wdc1, bdc1, wdc2, bdc2,
          wdl1, bdl1, wdl2, bdl2,
          wdh1, bdh1, wdh2, bdh2,
          out_ref, acc):
    g = pl.program_id(0)

    @pl.when(g == 0)
    def _init():
        acc[...] = jnp.zeros_like(acc)

    ids = ids_ref[...]
    iota_b = lax.broadcasted_iota(jnp.int32, (C, B), 1)
    onehot = (ids == iota_b).astype(_F32)

    if G > T // C:
        dropped = ids == ids_prev_ref[...]
        keep = jnp.where(g >= T // C, 1.0 - dropped.astype(_F32),
                         jnp.ones((C, 1), _F32))
    else:
        keep = jnp.ones((C, 1), _F32)

    a = jnp.maximum(_dot(feats_ref[...], wb1[...]) + bb1[...], 0.0)
    y = jnp.concatenate([a, coords_ref[...], jnp.ones((C, 1), _F32)],
                        axis=1) * keep
    acc[...] += _dotT(onehot, y)

    @pl.when(g == G - 1)
    def _final():
        n = acc[:, D + 3:D + 4]
        inv = 1.0 / jnp.maximum(n, 1.0)
        pt = (_dot(acc[:, :D], wb2[...]) + n * bb2[...]) * inv
        pc = acc[:, D:D + 3] * inv

        def head(w1, b1, w2, b2):
            h = jnp.maximum(_dot(pt, w1[0:D, :]) + _dot(pc, w1[D:D + 3, :])
                            + b1[...], 0.0)
            return _dot(h, w2[...]) + b2[...]

        ml = head(wm1, bm1, wm2, bm2)
        mx = jnp.max(ml, axis=-1, keepdims=True)
        ex = jnp.exp(ml - mx)
        probs = jnp.maximum(ex / jnp.sum(ex, axis=-1, keepdims=True), 1e-6)
        p_cont = probs[:, 0:2].sum(-1, keepdims=True)
        p_uncont = probs[:, 2:4].sum(-1, keepdims=True) + probs[:, 5:6]
        e_cont = head(wec1, bec1, wec2, bec2)
        e_uncont = head(weu1, beu1, weu2, beu2)
        energy = p_cont * e_cont + p_uncont * e_uncont
        p_cas = probs[:, 0:1]
        p_track = probs[:, 1:4].sum(-1, keepdims=True) + probs[:, 5:6]
        gate = 1.0 / (1.0 + jnp.exp(LOG_THRESH - energy[:, 0:1]))
        d_cas = head(wdc1, bdc1, wdc2, bdc2)
        d_low = head(wdl1, bdl1, wdl2, bdl2)
        d_high = head(wdh1, bdh1, wdh2, bdh2)
        dirp = (p_cas * d_cas + p_track * (1.0 - gate) * d_low
                + p_track * gate * d_high)
        out_ref[...] = jnp.concatenate([ml, energy, dirp], axis=-1)


def _chunk_spec(width):
    return pl.BlockSpec((C, width), lambda g: (g, 0))


def _const_spec(shape):
    return pl.BlockSpec(shape, lambda g: tuple(0 for _ in shape))


@functools.partial(jax.jit, static_argnames=("interpret",))
def _run(coords, features, ids2, *weights, interpret=False):
    wspecs = [_const_spec(w.shape) for w in weights]
    prev_spec = pl.BlockSpec((C, 1), lambda g: (jnp.maximum(g - T // C, 0), 0))
    return pl.pallas_call(
        _body,
        grid=(G,),
        in_specs=[_chunk_spec(F_IN), _chunk_spec(3), _chunk_spec(1), prev_spec]
        + wspecs,
        out_specs=_const_spec((B, 11)),
        out_shape=jax.ShapeDtypeStruct((B, 11), _F32),
        scratch_shapes=[pltpu.VMEM((B, D + 4), _F32)],
        interpret=interpret,
    )(features, coords, ids2, ids2, *weights)


def kernel(coords, features, batch_ids, Wb1, bb1, Wb2, bb2, Wm1, bm1, Wm2, bm2,
           Wec1, bec1, Wec2, bec2, Weu1, beu1, Weu2, beu2,
           Wdc1, bdc1, Wdc2, bdc2, Wdl1, bdl1, Wdl2, bdl2,
           Wdh1, bdh1, Wdh2, bdh2, interpret=False):
    ids2 = batch_ids.astype(jnp.int32).reshape(P, 1)
    return _run(coords, features, ids2,
                Wb1, bb1, Wb2, bb2, Wm1, bm1, Wm2, bm2,
                Wec1, bec1, Wec2, bec2, Weu1, beu1, Weu2, beu2,
                Wdc1, bdc1, Wdc2, bdc2, Wdl1, bdl1, Wdl2, bdl2,
                Wdh1, bdh1, Wdh2, bdh2, interpret=interpret)


# speedup vs baseline: 4.9408x; 1.1033x over previous
import functools

import jax
import jax.numpy as jnp
from jax import lax
from jax.experimental import pallas as pl
from jax.experimental.pallas import tpu as pltpu

P = 32768
B = 16
T = 4096
F_IN = 16
D = 128
H = 256
N_MORPH = 6
LOG_THRESH = 4.0

C = 4096
G = P // C

_F32 = jnp.float32
_HIGH = lax.Precision.HIGHEST


def _dotT(x, y):
    return lax.dot_general(x, y, (((0,), (0,)), ((), ())),
                           precision=_HIGH, preferred_element_type=_F32)


def _dot(x, y):
    return lax.dot_general(x, y, (((1,), (0,)), ((), ())),
                           precision=_HIGH, preferred_element_type=_F32)


def _body(feats_ref, coords_ref, ids_ref, ids_prev_ref,
          wb1, bb1, wb2, bb2,
          wm1, bm1, wm2, bm2,
          wec1, bec1, wec2, bec2,
          weu1, beu1, weu2, beu2,
          wdc1, bdc1, wdc2, bdc2,
          wdl1, bdl1, wdl2, bdl2,
          wdh1, bdh1, wdh2, bdh2,
          out_ref, acc):
    g = pl.program_id(0)

    @pl.when(g == 0)
    def _init():
        acc[...] = jnp.zeros_like(acc)

    ids = ids_ref[...].reshape(C, 1)
    iota_b = lax.broadcasted_iota(jnp.int32, (C, B), 1)
    onehot = (ids == iota_b).astype(_F32)

    if G > T // C:
        dropped = ids == ids_prev_ref[...].reshape(C, 1)
        keep = jnp.where(g >= T // C, 1.0 - dropped.astype(_F32),
                         jnp.ones((C, 1), _F32))
    else:
        keep = jnp.ones((C, 1), _F32)

    a = jnp.maximum(_dot(feats_ref[...], wb1[...]) + bb1[...], 0.0)
    y = jnp.concatenate([a, coords_ref[...], jnp.ones((C, 1), _F32)],
                        axis=1) * keep
    acc[...] += _dotT(onehot, y)

    @pl.when(g == G - 1)
    def _final():
        n = acc[:, D + 3:D + 4]
        inv = 1.0 / jnp.maximum(n, 1.0)
        pt = (_dot(acc[:, :D], wb2[...]) + n * bb2[...]) * inv
        pc = acc[:, D:D + 3] * inv

        def head(w1, b1, w2, b2):
            h = jnp.maximum(_dot(pt, w1[0:D, :]) + _dot(pc, w1[D:D + 3, :])
                            + b1[...], 0.0)
            return _dot(h, w2[...]) + b2[...]

        ml = head(wm1, bm1, wm2, bm2)
        mx = jnp.max(ml, axis=-1, keepdims=True)
        ex = jnp.exp(ml - mx)
        probs = jnp.maximum(ex / jnp.sum(ex, axis=-1, keepdims=True), 1e-6)
        p_cont = probs[:, 0:2].sum(-1, keepdims=True)
        p_uncont = probs[:, 2:4].sum(-1, keepdims=True) + probs[:, 5:6]
        e_cont = head(wec1, bec1, wec2, bec2)
        e_uncont = head(weu1, beu1, weu2, beu2)
        energy = p_cont * e_cont + p_uncont * e_uncont
        p_cas = probs[:, 0:1]
        p_track = probs[:, 1:4].sum(-1, keepdims=True) + probs[:, 5:6]
        gate = 1.0 / (1.0 + jnp.exp(LOG_THRESH - energy[:, 0:1]))
        d_cas = head(wdc1, bdc1, wdc2, bdc2)
        d_low = head(wdl1, bdl1, wdl2, bdl2)
        d_high = head(wdh1, bdh1, wdh2, bdh2)
        dirp = (p_cas * d_cas + p_track * (1.0 - gate) * d_low
                + p_track * gate * d_high)
        out_ref[...] = jnp.concatenate([ml, energy, dirp], axis=-1)


def _chunk_spec(width):
    return pl.BlockSpec((C, width), lambda g: (g, 0))


def _const_spec(shape):
    return pl.BlockSpec(shape, lambda g: tuple(0 for _ in shape))


@functools.partial(jax.jit, static_argnames=("interpret",))
def _run(coords, features, ids2, *weights, interpret=False):
    wspecs = [_const_spec(w.shape) for w in weights]
    ids_spec = pl.BlockSpec((C,), lambda g: (g,))
    prev_spec = pl.BlockSpec((C,), lambda g: (jnp.maximum(g - T // C, 0),))
    return pl.pallas_call(
        _body,
        grid=(G,),
        in_specs=[_chunk_spec(F_IN), _chunk_spec(3), ids_spec, prev_spec]
        + wspecs,
        out_specs=_const_spec((B, 11)),
        out_shape=jax.ShapeDtypeStruct((B, 11), _F32),
        scratch_shapes=[pltpu.VMEM((B, D + 4), _F32)],
        interpret=interpret,
    )(features, coords, ids2, ids2, *weights)


def kernel(coords, features, batch_ids, Wb1, bb1, Wb2, bb2, Wm1, bm1, Wm2, bm2,
           Wec1, bec1, Wec2, bec2, Weu1, beu1, Weu2, beu2,
           Wdc1, bdc1, Wdc2, bdc2, Wdl1, bdl1, Wdl2, bdl2,
           Wdh1, bdh1, Wdh2, bdh2, interpret=False):
    ids2 = batch_ids.astype(jnp.int32)
    return _run(coords, features, ids2,
                Wb1, bb1, Wb2, bb2, Wm1, bm1, Wm2, bm2,
                Wec1, bec1, Wec2, bec2, Weu1, beu1, Weu2, beu2,
                Wdc1, bdc1, Wdc2, bdc2, Wdl1, bdl1, Wdl2, bdl2,
                Wdh1, bdh1, Wdh2, bdh2, interpret=interpret)


# speedup vs baseline: 8.3385x; 1.6877x over previous
import functools

import jax
import jax.numpy as jnp
from jax import lax
from jax.experimental import pallas as pl
from jax.experimental.pallas import tpu as pltpu

P = 32768
B = 16
T = 4096
F_IN = 16
D = 128
H = 256
N_MORPH = 6
LOG_THRESH = 4.0

C = 4096
G = P // C

_F32 = jnp.float32
_HIGH = lax.Precision.HIGHEST


def _dotT(x, y, precision=None):
    return lax.dot_general(x, y, (((0,), (0,)), ((), ())),
                           precision=precision, preferred_element_type=_F32)


def _dot(x, y, precision=_HIGH):
    return lax.dot_general(x, y, (((1,), (0,)), ((), ())),
                           precision=precision, preferred_element_type=_F32)


def _body(feats_ref, coords_ref, ids_ref, ids_prev_ref,
          wb1, bb1, wb2, bb2,
          wm1, bm1, wm2, bm2,
          wec1, bec1, wec2, bec2,
          weu1, beu1, weu2, beu2,
          wdc1, bdc1, wdc2, bdc2,
          wdl1, bdl1, wdl2, bdl2,
          wdh1, bdh1, wdh2, bdh2,
          out_ref, acc):
    g = pl.program_id(0)

    @pl.when(g == 0)
    def _init():
        acc[...] = jnp.zeros_like(acc)

    ids = ids_ref[...].reshape(C, 1)
    iota_b = lax.broadcasted_iota(jnp.int32, (C, B), 1)

    if G > T // C:
        kept = (ids != ids_prev_ref[...].reshape(C, 1)) | (g < T // C)
        om = ((ids == iota_b) & kept).astype(_F32)
    else:
        om = (ids == iota_b).astype(_F32)

    a = jnp.maximum(_dot(feats_ref[...], wb1[...], None) + bb1[...], 0.0)
    acc[:, :D] += _dotT(om, a)
    xe = jnp.concatenate([coords_ref[...], jnp.ones((C, 1), _F32)], axis=1)
    acc[:, D:] += _dotT(om, xe)

    @pl.when(g == G - 1)
    def _final():
        n = acc[:, D + 3:D + 4]
        inv = 1.0 / jnp.maximum(n, 1.0)
        pt = (_dot(acc[:, :D], wb2[...]) + n * bb2[...]) * inv
        pc = acc[:, D:D + 3] * inv

        def head(w1, b1, w2, b2):
            h = jnp.maximum(_dot(pt, w1[0:D, :]) + _dot(pc, w1[D:D + 3, :])
                            + b1[...], 0.0)
            return _dot(h, w2[...]) + b2[...]

        ml = head(wm1, bm1, wm2, bm2)
        mx = jnp.max(ml, axis=-1, keepdims=True)
        ex = jnp.exp(ml - mx)
        probs = jnp.maximum(ex / jnp.sum(ex, axis=-1, keepdims=True), 1e-6)
        p_cont = probs[:, 0:2].sum(-1, keepdims=True)
        p_uncont = probs[:, 2:4].sum(-1, keepdims=True) + probs[:, 5:6]
        e_cont = head(wec1, bec1, wec2, bec2)
        e_uncont = head(weu1, beu1, weu2, beu2)
        energy = p_cont * e_cont + p_uncont * e_uncont
        p_cas = probs[:, 0:1]
        p_track = probs[:, 1:4].sum(-1, keepdims=True) + probs[:, 5:6]
        gate = 1.0 / (1.0 + jnp.exp(LOG_THRESH - energy[:, 0:1]))
        d_cas = head(wdc1, bdc1, wdc2, bdc2)
        d_low = head(wdl1, bdl1, wdl2, bdl2)
        d_high = head(wdh1, bdh1, wdh2, bdh2)
        dirp = (p_cas * d_cas + p_track * (1.0 - gate) * d_low
                + p_track * gate * d_high)
        out_ref[...] = jnp.concatenate([ml, energy, dirp], axis=-1)


def _chunk_spec(width):
    return pl.BlockSpec((C, width), lambda g: (g, 0))


def _const_spec(shape):
    return pl.BlockSpec(shape, lambda g: tuple(0 for _ in shape))


@functools.partial(jax.jit, static_argnames=("interpret",))
def _run(coords, features, ids2, *weights, interpret=False):
    wspecs = [_const_spec(w.shape) for w in weights]
    ids_spec = pl.BlockSpec((C,), lambda g: (g,))
    prev_spec = pl.BlockSpec((C,), lambda g: (jnp.maximum(g - T // C, 0),))
    return pl.pallas_call(
        _body,
        grid=(G,),
        in_specs=[_chunk_spec(F_IN), _chunk_spec(3), ids_spec, prev_spec]
        + wspecs,
        out_specs=_const_spec((B, 11)),
        out_shape=jax.ShapeDtypeStruct((B, 11), _F32),
        scratch_shapes=[pltpu.VMEM((B, D + 4), _F32)],
        interpret=interpret,
    )(features, coords, ids2, ids2, *weights)


def kernel(coords, features, batch_ids, Wb1, bb1, Wb2, bb2, Wm1, bm1, Wm2, bm2,
           Wec1, bec1, Wec2, bec2, Weu1, beu1, Weu2, beu2,
           Wdc1, bdc1, Wdc2, bdc2, Wdl1, bdl1, Wdl2, bdl2,
           Wdh1, bdh1, Wdh2, bdh2, interpret=False):
    ids2 = batch_ids.astype(jnp.int32)
    return _run(coords, features, ids2,
                Wb1, bb1, Wb2, bb2, Wm1, bm1, Wm2, bm2,
                Wec1, bec1, Wec2, bec2, Weu1, beu1, Weu2, beu2,
                Wdc1, bdc1, Wdc2, bdc2, Wdl1, bdl1, Wdl2, bdl2,
                Wdh1, bdh1, Wdh2, bdh2, interpret=interpret)


# speedup vs baseline: 9.1800x; 1.1009x over previous
import functools

import jax
import jax.numpy as jnp
from jax import lax
from jax.experimental import pallas as pl
from jax.experimental.pallas import tpu as pltpu

P = 32768
B = 16
T = 4096
F_IN = 16
D = 128
H = 256
N_MORPH = 6
LOG_THRESH = 4.0

C = 4096
G = P // C

_F32 = jnp.float32
_HIGH = lax.Precision.HIGHEST


def _dotT(x, y, precision=None):
    return lax.dot_general(x, y, (((0,), (0,)), ((), ())),
                           precision=precision, preferred_element_type=_F32)


def _dot(x, y, precision=_HIGH):
    return lax.dot_general(x, y, (((1,), (0,)), ((), ())),
                           precision=precision, preferred_element_type=_F32)


def _body(feats_ref, coords_ref, ids_ref, ids_prev_ref,
          wb1, bb1, wb2, bb2,
          wm1, bm1, wm2, bm2,
          wec1, bec1, wec2, bec2,
          weu1, beu1, weu2, beu2,
          wdc1, bdc1, wdc2, bdc2,
          wdl1, bdl1, wdl2, bdl2,
          wdh1, bdh1, wdh2, bdh2,
          out_ref, acc):
    g = pl.program_id(0)

    @pl.when(g == 0)
    def _init():
        acc[...] = jnp.zeros_like(acc)

    ids = ids_ref[...]
    iota_b = lax.broadcasted_iota(jnp.int32, (B, C), 0)

    if G > T // C:
        kept = (ids != ids_prev_ref[...]) | (g < T // C)
        om = ((ids == iota_b) & kept).astype(_F32)
    else:
        om = (ids == iota_b).astype(_F32)

    a = jnp.maximum(_dot(feats_ref[...], wb1[...], None) + bb1[...], 0.0)
    acc[:, :D] += _dot(om, a, None)
    acc[:, D:D + 3] += _dot(om, coords_ref[...], None)
    acc[:, D + 3:] += jnp.sum(om, axis=1, keepdims=True)

    @pl.when(g == G - 1)
    def _final():
        n = acc[:, D + 3:D + 4]
        inv = 1.0 / jnp.maximum(n, 1.0)
        pt = (_dot(acc[:, :D], wb2[...]) + n * bb2[...]) * inv
        pc = acc[:, D:D + 3] * inv

        def head(w1, b1, w2, b2):
            h = jnp.maximum(_dot(pt, w1[0:D, :]) + _dot(pc, w1[D:D + 3, :])
                            + b1[...], 0.0)
            return _dot(h, w2[...]) + b2[...]

        ml = head(wm1, bm1, wm2, bm2)
        mx = jnp.max(ml, axis=-1, keepdims=True)
        ex = jnp.exp(ml - mx)
        probs = jnp.maximum(ex / jnp.sum(ex, axis=-1, keepdims=True), 1e-6)
        p_cont = probs[:, 0:2].sum(-1, keepdims=True)
        p_uncont = probs[:, 2:4].sum(-1, keepdims=True) + probs[:, 5:6]
        e_cont = head(wec1, bec1, wec2, bec2)
        e_uncont = head(weu1, beu1, weu2, beu2)
        energy = p_cont * e_cont + p_uncont * e_uncont
        p_cas = probs[:, 0:1]
        p_track = probs[:, 1:4].sum(-1, keepdims=True) + probs[:, 5:6]
        gate = 1.0 / (1.0 + jnp.exp(LOG_THRESH - energy[:, 0:1]))
        d_cas = head(wdc1, bdc1, wdc2, bdc2)
        d_low = head(wdl1, bdl1, wdl2, bdl2)
        d_high = head(wdh1, bdh1, wdh2, bdh2)
        dirp = (p_cas * d_cas + p_track * (1.0 - gate) * d_low
                + p_track * gate * d_high)
        out_ref[...] = jnp.concatenate([ml, energy, dirp], axis=-1)


def _chunk_spec(width):
    return pl.BlockSpec((C, width), lambda g: (g, 0))


def _const_spec(shape):
    return pl.BlockSpec(shape, lambda g: tuple(0 for _ in shape))


@functools.partial(jax.jit, static_argnames=("interpret",))
def _run(coords, features, ids2, *weights, interpret=False):
    wspecs = [_const_spec(w.shape) for w in weights]
    ids_spec = pl.BlockSpec((1, C), lambda g: (0, g))
    prev_spec = pl.BlockSpec((1, C), lambda g: (0, jnp.maximum(g - T // C, 0)))
    return pl.pallas_call(
        _body,
        grid=(G,),
        in_specs=[_chunk_spec(F_IN), _chunk_spec(3), ids_spec, prev_spec]
        + wspecs,
        out_specs=_const_spec((B, 11)),
        out_shape=jax.ShapeDtypeStruct((B, 11), _F32),
        scratch_shapes=[pltpu.VMEM((B, D + 4), _F32)],
        interpret=interpret,
    )(features, coords, ids2, ids2, *weights)


def kernel(coords, features, batch_ids, Wb1, bb1, Wb2, bb2, Wm1, bm1, Wm2, bm2,
           Wec1, bec1, Wec2, bec2, Weu1, beu1, Weu2, beu2,
           Wdc1, bdc1, Wdc2, bdc2, Wdl1, bdl1, Wdl2, bdl2,
           Wdh1, bdh1, Wdh2, bdh2, interpret=False):
    ids2 = batch_ids.astype(jnp.int32).reshape(1, P)
    return _run(coords, features, ids2,
                Wb1, bb1, Wb2, bb2, Wm1, bm1, Wm2, bm2,
                Wec1, bec1, Wec2, bec2, Weu1, beu1, Weu2, beu2,
                Wdc1, bdc1, Wdc2, bdc2, Wdl1, bdl1, Wdl2, bdl2,
                Wdh1, bdh1, Wdh2, bdh2, interpret=interpret)


# speedup vs baseline: 9.2118x; 1.0035x over previous
import functools

import jax
import jax.numpy as jnp
from jax import lax
from jax.experimental import pallas as pl
from jax.experimental.pallas import tpu as pltpu

P = 32768
B = 16
T = 4096
F_IN = 16
D = 128
H = 256
N_MORPH = 6
LOG_THRESH = 4.0

C = 4096
G = P // C

_F32 = jnp.float32
_BF16 = jnp.bfloat16
_HIGH = lax.Precision.HIGHEST


def _dotT(x, y, precision=None):
    return lax.dot_general(x, y, (((0,), (0,)), ((), ())),
                           precision=precision, preferred_element_type=_F32)


def _dot(x, y, precision=_HIGH):
    return lax.dot_general(x, y, (((1,), (0,)), ((), ())),
                           precision=precision, preferred_element_type=_F32)


def _body(feats_ref, coords_ref, ids_ref, ids_prev_ref,
          wb1, bb1, wb2, bb2,
          wm1, bm1, wm2, bm2,
          wec1, bec1, wec2, bec2,
          weu1, beu1, weu2, beu2,
          wdc1, bdc1, wdc2, bdc2,
          wdl1, bdl1, wdl2, bdl2,
          wdh1, bdh1, wdh2, bdh2,
          out_ref, acc):
    g = pl.program_id(0)

    @pl.when(g == 0)
    def _init():
        acc[...] = jnp.zeros_like(acc)

    ids = ids_ref[...]
    iota_b = lax.broadcasted_iota(jnp.int32, (B, C), 0)

    if G > T // C:
        kept = (ids != ids_prev_ref[...]) | (g < T // C)
        omb = (ids == iota_b) & kept
    else:
        omb = ids == iota_b
    om = omb.astype(_BF16)

    fb = feats_ref[...].astype(_BF16)
    a = jnp.maximum(_dot(fb, wb1[...].astype(_BF16), None) + bb1[...], 0.0)
    acc[:, :D] += _dot(om, a.astype(_BF16), None)
    acc[:, D:D + 3] += _dot(om, coords_ref[...].astype(_BF16), None)
    acc[:, D + 3:] += jnp.sum(omb.astype(_F32), axis=1, keepdims=True)

    @pl.when(g == G - 1)
    def _final():
        n = acc[:, D + 3:D + 4]
        inv = 1.0 / jnp.maximum(n, 1.0)
        pt = (_dot(acc[:, :D], wb2[...]) + n * bb2[...]) * inv
        pc = acc[:, D:D + 3] * inv

        def head(w1, b1, w2, b2):
            h = jnp.maximum(_dot(pt, w1[0:D, :]) + _dot(pc, w1[D:D + 3, :])
                            + b1[...], 0.0)
            return _dot(h, w2[...]) + b2[...]

        ml = head(wm1, bm1, wm2, bm2)
        mx = jnp.max(ml, axis=-1, keepdims=True)
        ex = jnp.exp(ml - mx)
        probs = jnp.maximum(ex / jnp.sum(ex, axis=-1, keepdims=True), 1e-6)
        p_cont = probs[:, 0:2].sum(-1, keepdims=True)
        p_uncont = probs[:, 2:4].sum(-1, keepdims=True) + probs[:, 5:6]
        e_cont = head(wec1, bec1, wec2, bec2)
        e_uncont = head(weu1, beu1, weu2, beu2)
        energy = p_cont * e_cont + p_uncont * e_uncont
        p_cas = probs[:, 0:1]
        p_track = probs[:, 1:4].sum(-1, keepdims=True) + probs[:, 5:6]
        gate = 1.0 / (1.0 + jnp.exp(LOG_THRESH - energy[:, 0:1]))
        d_cas = head(wdc1, bdc1, wdc2, bdc2)
        d_low = head(wdl1, bdl1, wdl2, bdl2)
        d_high = head(wdh1, bdh1, wdh2, bdh2)
        dirp = (p_cas * d_cas + p_track * (1.0 - gate) * d_low
                + p_track * gate * d_high)
        out_ref[...] = jnp.concatenate([ml, energy, dirp], axis=-1)


def _chunk_spec(width):
    return pl.BlockSpec((C, width), lambda g: (g, 0))


def _const_spec(shape):
    return pl.BlockSpec(shape, lambda g: tuple(0 for _ in shape))


@functools.partial(jax.jit, static_argnames=("interpret",))
def _run(coords, features, ids2, *weights, interpret=False):
    wspecs = [_const_spec(w.shape) for w in weights]
    ids_spec = pl.BlockSpec((1, C), lambda g: (0, g))
    prev_spec = pl.BlockSpec((1, C), lambda g: (0, jnp.maximum(g - T // C, 0)))
    return pl.pallas_call(
        _body,
        grid=(G,),
        in_specs=[_chunk_spec(F_IN), _chunk_spec(3), ids_spec, prev_spec]
        + wspecs,
        out_specs=_const_spec((B, 11)),
        out_shape=jax.ShapeDtypeStruct((B, 11), _F32),
        scratch_shapes=[pltpu.VMEM((B, D + 4), _F32)],
        interpret=interpret,
    )(features, coords, ids2, ids2, *weights)


def kernel(coords, features, batch_ids, Wb1, bb1, Wb2, bb2, Wm1, bm1, Wm2, bm2,
           Wec1, bec1, Wec2, bec2, Weu1, beu1, Weu2, beu2,
           Wdc1, bdc1, Wdc2, bdc2, Wdl1, bdl1, Wdl2, bdl2,
           Wdh1, bdh1, Wdh2, bdh2, interpret=False):
    ids2 = batch_ids.astype(jnp.int32).reshape(1, P)
    return _run(coords, features, ids2,
                Wb1, bb1, Wb2, bb2, Wm1, bm1, Wm2, bm2,
                Wec1, bec1, Wec2, bec2, Weu1, beu1, Weu2, beu2,
                Wdc1, bdc1, Wdc2, bdc2, Wdl1, bdl1, Wdl2, bdl2,
                Wdh1, bdh1, Wdh2, bdh2, interpret=interpret)


# speedup vs baseline: 9.4347x; 1.0242x over previous
import functools

import jax
import jax.numpy as jnp
from jax import lax
from jax.experimental import pallas as pl
from jax.experimental.pallas import tpu as pltpu

P = 32768
B = 16
T = 4096
F_IN = 16
D = 128
H = 256
N_MORPH = 6
LOG_THRESH = 4.0

C = 8192
G = P // C

_F32 = jnp.float32
_BF16 = jnp.bfloat16
_HIGH = lax.Precision.HIGHEST


def _dotT(x, y, precision=None):
    return lax.dot_general(x, y, (((0,), (0,)), ((), ())),
                           precision=precision, preferred_element_type=_F32)


def _dot(x, y, precision=_HIGH):
    return lax.dot_general(x, y, (((1,), (0,)), ((), ())),
                           precision=precision, preferred_element_type=_F32)


def _body(feats_ref, coords_ref, ids_ref, ids_prev_ref,
          wb1, bb1, wb2, bb2,
          wm1, bm1, wm2, bm2,
          wec1, bec1, wec2, bec2,
          weu1, beu1, weu2, beu2,
          wdc1, bdc1, wdc2, bdc2,
          wdl1, bdl1, wdl2, bdl2,
          wdh1, bdh1, wdh2, bdh2,
          out_ref, acc):
    g = pl.program_id(0)

    @pl.when(g == 0)
    def _init():
        acc[...] = jnp.zeros_like(acc)

    ids = ids_ref[...]
    iota_b = lax.broadcasted_iota(jnp.int32, (B, C), 0)

    kept_lo = (ids[:, :T] != ids_prev_ref[...][:, C - T:]) | (g == 0)
    kept_hi = ids[:, T:] != ids[:, :C - T]
    kept = jnp.concatenate([kept_lo, kept_hi], axis=1)
    omb = (ids == iota_b) & kept
    om = omb.astype(_BF16)

    fb = feats_ref[...].astype(_BF16)
    a = jnp.maximum(_dot(fb, wb1[...].astype(_BF16), None) + bb1[...], 0.0)
    acc[:, :D] += _dot(om, a.astype(_BF16), None)
    acc[:, D:D + 3] += _dot(om, coords_ref[...].astype(_BF16), None)
    acc[:, D + 3:] += jnp.sum(omb.astype(_F32), axis=1, keepdims=True)

    @pl.when(g == G - 1)
    def _final():
        n = acc[:, D + 3:D + 4]
        inv = 1.0 / jnp.maximum(n, 1.0)
        pt = (_dot(acc[:, :D], wb2[...]) + n * bb2[...]) * inv
        pc = acc[:, D:D + 3] * inv

        def head(w1, b1, w2, b2):
            h = jnp.maximum(_dot(pt, w1[0:D, :]) + _dot(pc, w1[D:D + 3, :])
                            + b1[...], 0.0)
            return _dot(h, w2[...]) + b2[...]

        ml = head(wm1, bm1, wm2, bm2)
        mx = jnp.max(ml, axis=-1, keepdims=True)
        ex = jnp.exp(ml - mx)
        probs = jnp.maximum(ex / jnp.sum(ex, axis=-1, keepdims=True), 1e-6)
        p_cont = probs[:, 0:2].sum(-1, keepdims=True)
        p_uncont = probs[:, 2:4].sum(-1, keepdims=True) + probs[:, 5:6]
        e_cont = head(wec1, bec1, wec2, bec2)
        e_uncont = head(weu1, beu1, weu2, beu2)
        energy = p_cont * e_cont + p_uncont * e_uncont
        p_cas = probs[:, 0:1]
        p_track = probs[:, 1:4].sum(-1, keepdims=True) + probs[:, 5:6]
        gate = 1.0 / (1.0 + jnp.exp(LOG_THRESH - energy[:, 0:1]))
        d_cas = head(wdc1, bdc1, wdc2, bdc2)
        d_low = head(wdl1, bdl1, wdl2, bdl2)
        d_high = head(wdh1, bdh1, wdh2, bdh2)
        dirp = (p_cas * d_cas + p_track * (1.0 - gate) * d_low
                + p_track * gate * d_high)
        out_ref[...] = jnp.concatenate([ml, energy, dirp], axis=-1)


def _chunk_spec(width):
    return pl.BlockSpec((C, width), lambda g: (g, 0))


def _const_spec(shape):
    return pl.BlockSpec(shape, lambda g: tuple(0 for _ in shape))


@functools.partial(jax.jit, static_argnames=("interpret",))
def _run(coords, features, ids2, *weights, interpret=False):
    wspecs = [_const_spec(w.shape) for w in weights]
    ids_spec = pl.BlockSpec((1, C), lambda g: (0, g))
    prev_spec = pl.BlockSpec((1, C), lambda g: (0, jnp.maximum(g - 1, 0)))
    return pl.pallas_call(
        _body,
        grid=(G,),
        in_specs=[_chunk_spec(F_IN), _chunk_spec(3), ids_spec, prev_spec]
        + wspecs,
        out_specs=_const_spec((B, 11)),
        out_shape=jax.ShapeDtypeStruct((B, 11), _F32),
        scratch_shapes=[pltpu.VMEM((B, D + 4), _F32)],
        interpret=interpret,
    )(features, coords, ids2, ids2, *weights)


def kernel(coords, features, batch_ids, Wb1, bb1, Wb2, bb2, Wm1, bm1, Wm2, bm2,
           Wec1, bec1, Wec2, bec2, Weu1, beu1, Weu2, beu2,
           Wdc1, bdc1, Wdc2, bdc2, Wdl1, bdl1, Wdl2, bdl2,
           Wdh1, bdh1, Wdh2, bdh2, interpret=False):
    ids2 = batch_ids.astype(jnp.int32).reshape(1, P)
    return _run(coords, features, ids2,
                Wb1, bb1, Wb2, bb2, Wm1, bm1, Wm2, bm2,
                Wec1, bec1, Wec2, bec2, Weu1, beu1, Weu2, beu2,
                Wdc1, bdc1, Wdc2, bdc2, Wdl1, bdl1, Wdl2, bdl2,
                Wdh1, bdh1, Wdh2, bdh2, interpret=interpret)
